# baseline (speedup 1.0000x reference)
import functools

import jax
import jax.numpy as jnp
from jax import lax
from jax.experimental import pallas as pl
from jax.experimental.pallas import tpu as pltpu
from jax.experimental.pallas import tpu_sc as plsc

B = 4
H = 8192
F = 1024
N = H * F
NC, NS = 2, 16
NW = NC * NS
PER_W = N // NW
CH = 16384
NCH = PER_W // CH
VEC = 16


def _sc_body(x_hbm, w_hbm, o_hbm, wbuf, xbuf):
    wid = lax.axis_index("s") * NC + lax.axis_index("c")
    base = wid * PER_W

    def chunk_loop(c, carry):
        off = base + c * CH
        pltpu.sync_copy(w_hbm.at[pl.ds(off, CH)], wbuf)

        def batch_loop(b, carry2):
            pltpu.sync_copy(x_hbm.at[b, pl.ds(off, CH)], xbuf)

            def vec_loop(i, carry3):
                s = pl.ds(i * VEC, VEC)
                xbuf[s] = xbuf[s] + wbuf[s]
                return carry3

            lax.fori_loop(0, CH // VEC, vec_loop, 0, unroll=8)
            pltpu.sync_copy(xbuf, o_hbm.at[b, pl.ds(off, CH)])
            return carry2

        lax.fori_loop(0, B, batch_loop, 0)
        return carry

    lax.fori_loop(0, NCH, chunk_loop, 0)


@jax.jit
def kernel(x, embed_weight):
    x2 = x.reshape(B, N)
    w1 = embed_weight.reshape(N)
    mesh = plsc.VectorSubcoreMesh(core_axis_name="c", subcore_axis_name="s")
    out = pl.kernel(
        _sc_body,
        out_type=jax.ShapeDtypeStruct((B, N), jnp.float32),
        mesh=mesh,
        scratch_types=[
            pltpu.VMEM((CH,), jnp.float32),
            pltpu.VMEM((CH,), jnp.float32),
        ],
    )(x2, w1)
    return out.reshape(B, H, F)


# speedup vs baseline: 1.2518x; 1.2518x over previous
import jax
import jax.numpy as jnp
from jax import lax
from jax.experimental import pallas as pl
from jax.experimental.pallas import tpu as pltpu
from jax.experimental.pallas import tpu_sc as plsc

B = 4
H = 8192
F = 1024
N = H * F
NC, NS = 2, 16
NW = NC * NS
PER_W = N // NW
CH = 8192
NCH = PER_W // CH
VEC = 16


def _sc_body(x_hbm, w_hbm, o_hbm,
             w0, x00, x01, x02, x03,
             w1, x10, x11, x12, x13,
             sem_in0, sem_in1, sem_out0, sem_out1):
    wid = lax.axis_index("s") * NC + lax.axis_index("c")
    base = wid * PER_W

    sets = ((w0, (x00, x01, x02, x03), sem_in0, sem_out0),
            (w1, (x10, x11, x12, x13), sem_in1, sem_out1))

    def fire_in(p, c):
        wbuf, xbufs, sem_in, _ = sets[p]
        off = base + c * CH
        pltpu.async_copy(w_hbm.at[pl.ds(off, CH)], wbuf, sem_in)
        for b in range(B):
            pltpu.async_copy(x_hbm.at[b, pl.ds(off, CH)], xbufs[b], sem_in)

    def drain_in(p, c):
        wbuf, xbufs, sem_in, _ = sets[p]
        off = base + c * CH
        pltpu.make_async_copy(w_hbm.at[pl.ds(off, CH)], wbuf, sem_in).wait()
        for b in range(B):
            pltpu.make_async_copy(
                x_hbm.at[b, pl.ds(off, CH)], xbufs[b], sem_in).wait()

    def drain_out(p, c):
        _, xbufs, _, sem_out = sets[p]
        off = base + c * CH
        for b in range(B):
            pltpu.make_async_copy(
                xbufs[b], o_hbm.at[b, pl.ds(off, CH)], sem_out).wait()

    fire_in(0, 0)

    def chunk_pair(c2, carry):
        for p in (0, 1):
            wbuf, xbufs, _, sem_out = sets[p]
            c = 2 * c2 + p
            off = base + c * CH

            @pl.when(c >= 1)
            def _():
                drain_out(1 - p, c - 1)

            @pl.when(c + 1 < NCH)
            def _():
                fire_in(1 - p, c + 1)

            drain_in(p, c)

            for b in range(B):
                xb = xbufs[b]

                def vec_loop(i, carry3, xb=xb):
                    s = pl.ds(i * VEC, VEC)
                    xb[s] = xb[s] + wbuf[s]
                    return carry3

                lax.fori_loop(0, CH // VEC, vec_loop, 0, unroll=8)
                pltpu.async_copy(xb, o_hbm.at[b, pl.ds(off, CH)], sem_out)
        return carry

    lax.fori_loop(0, NCH // 2, chunk_pair, 0)
    drain_out((NCH - 1) % 2, NCH - 1)


@jax.jit
def kernel(x, embed_weight):
    x2 = x.reshape(B, N)
    w1 = embed_weight.reshape(N)
    mesh = plsc.VectorSubcoreMesh(core_axis_name="c", subcore_axis_name="s")
    out = pl.kernel(
        _sc_body,
        out_type=jax.ShapeDtypeStruct((B, N), jnp.float32),
        mesh=mesh,
        scratch_types=(
            [pltpu.VMEM((CH,), jnp.float32)] * 10
            + [pltpu.SemaphoreType.DMA] * 4
        ),
    )(x2, w1)
    return out.reshape(B, H, F)


# speedup vs baseline: 3.3260x; 2.6569x over previous
import jax
import jax.numpy as jnp
from jax import lax
from jax.experimental import pallas as pl
from jax.experimental.pallas import tpu as pltpu
from jax.experimental.pallas import tpu_sc as plsc

B = 4
H = 8192
F = 1024
NC, NS = 2, 16
NW = NC * NS
ROWS_W = H // NW
CR = 8
NCH = ROWS_W // CR
VEC = 16
NV = F // VEC


def _sc_body(x_hbm, w_hbm, o_hbm,
             w0, x00, x01, x02, x03,
             w1, x10, x11, x12, x13,
             sem_in0, sem_in1, sem_out0, sem_out1):
    wid = lax.axis_index("s") * NC + lax.axis_index("c")
    base = wid * ROWS_W

    sets = ((w0, (x00, x01, x02, x03), sem_in0, sem_out0),
            (w1, (x10, x11, x12, x13), sem_in1, sem_out1))

    def fire_in(p, c):
        wbuf, xbufs, sem_in, _ = sets[p]
        r0 = base + c * CR
        pltpu.async_copy(w_hbm.at[pl.ds(r0, CR), :], wbuf, sem_in)
        for b in range(B):
            pltpu.async_copy(x_hbm.at[b, pl.ds(r0, CR), :], xbufs[b], sem_in)

    def drain_in(p, c):
        wbuf, xbufs, sem_in, _ = sets[p]
        r0 = base + c * CR
        pltpu.make_async_copy(w_hbm.at[pl.ds(r0, CR), :], wbuf, sem_in).wait()
        for b in range(B):
            pltpu.make_async_copy(
                x_hbm.at[b, pl.ds(r0, CR), :], xbufs[b], sem_in).wait()

    def drain_out(p, c):
        _, xbufs, _, sem_out = sets[p]
        r0 = base + c * CR
        for b in range(B):
            pltpu.make_async_copy(
                xbufs[b], o_hbm.at[b, pl.ds(r0, CR), :], sem_out).wait()

    fire_in(0, 0)

    def chunk_pair(c2, carry):
        for p in (0, 1):
            wbuf, xbufs, _, sem_out = sets[p]
            c = 2 * c2 + p
            r0 = base + c * CR

            @pl.when(c >= 1)
            def _():
                drain_out(1 - p, c - 1)

            @pl.when(c + 1 < NCH)
            def _():
                fire_in(1 - p, c + 1)

            drain_in(p, c)

            for b in range(B):
                xb = xbufs[b]

                def vec_loop(i, carry3, xb=xb):
                    s = pl.ds(i * VEC, VEC)
                    for r in range(CR):
                        xb[r, s] = xb[r, s] + wbuf[r, s]
                    return carry3

                lax.fori_loop(0, NV, vec_loop, 0, unroll=2)
                pltpu.async_copy(xb, o_hbm.at[b, pl.ds(r0, CR), :], sem_out)
        return carry

    lax.fori_loop(0, NCH // 2, chunk_pair, 0)
    drain_out((NCH - 1) % 2, NCH - 1)


@jax.jit
def kernel(x, embed_weight):
    mesh = plsc.VectorSubcoreMesh(core_axis_name="c", subcore_axis_name="s")
    return pl.kernel(
        _sc_body,
        out_type=jax.ShapeDtypeStruct((B, H, F), jnp.float32),
        mesh=mesh,
        scratch_types=(
            [pltpu.VMEM((CR, F), jnp.float32)] * 10
            + [pltpu.SemaphoreType.DMA] * 4
        ),
    )(x, embed_weight)


# speedup vs baseline: 5.3652x; 1.6131x over previous
import jax
import jax.numpy as jnp
from jax import lax
from jax.experimental import pallas as pl
from jax.experimental.pallas import tpu as pltpu
from jax.experimental.pallas import tpu_sc as plsc

B = 4
H = 8192
F = 1024
NC, NS = 2, 16
NW = NC * NS
ROWS_W = H // NW
CR = 8
NCH = ROWS_W // CR
VEC = 16
NV = F // VEC


def _sc_body(x_hbm, w_hbm, o_hbm,
             w0, x00, x01, x02, x03,
             w1, x10, x11, x12, x13,
             sem_in0, sem_in1, sem_out0, sem_out1):
    wid = lax.axis_index("s") * NC + lax.axis_index("c")
    base = wid * ROWS_W

    sets = ((w0, (x00, x01, x02, x03), sem_in0, sem_out0),
            (w1, (x10, x11, x12, x13), sem_in1, sem_out1))

    def fire_in(p, c):
        wbuf, xbufs, sem_in, _ = sets[p]
        r0 = base + c * CR
        pltpu.async_copy(w_hbm.at[pl.ds(r0, CR), :], wbuf, sem_in)
        for b in range(B):
            pltpu.async_copy(x_hbm.at[b, pl.ds(r0, CR), :], xbufs[b], sem_in)

    def drain_in(p, c):
        wbuf, xbufs, sem_in, _ = sets[p]
        r0 = base + c * CR
        pltpu.make_async_copy(w_hbm.at[pl.ds(r0, CR), :], wbuf, sem_in).wait()
        for b in range(B):
            pltpu.make_async_copy(
                x_hbm.at[b, pl.ds(r0, CR), :], xbufs[b], sem_in).wait()

    def drain_out(p, c):
        _, xbufs, _, sem_out = sets[p]
        r0 = base + c * CR
        for b in range(B):
            pltpu.make_async_copy(
                xbufs[b], o_hbm.at[b, pl.ds(r0, CR), :], sem_out).wait()

    fire_in(0, 0)

    def chunk_pair(c2, carry):
        for p in (0, 1):
            wbuf, xbufs, _, sem_out = sets[p]
            c = 2 * c2 + p
            r0 = base + c * CR

            @pl.when(c >= 1)
            def _():
                drain_out(1 - p, c - 1)

            @pl.when(c + 1 < NCH)
            def _():
                fire_in(1 - p, c + 1)

            drain_in(p, c)

            for b in range(B):
                xb = xbufs[b]

                def vec_loop(i, xb=xb):
                    s = pl.ds(i * VEC, VEC)
                    for r in range(CR):
                        xb[r, s] = xb[r, s] + wbuf[r, s]

                plsc.parallel_loop(0, NV, unroll=4)(vec_loop)
                pltpu.async_copy(xb, o_hbm.at[b, pl.ds(r0, CR), :], sem_out)
        return carry

    lax.fori_loop(0, NCH // 2, chunk_pair, 0)
    drain_out((NCH - 1) % 2, NCH - 1)


@jax.jit
def kernel(x, embed_weight):
    mesh = plsc.VectorSubcoreMesh(core_axis_name="c", subcore_axis_name="s")
    return pl.kernel(
        _sc_body,
        out_type=jax.ShapeDtypeStruct((B, H, F), jnp.float32),
        mesh=mesh,
        scratch_types=(
            [pltpu.VMEM((CR, F), jnp.float32)] * 10
            + [pltpu.SemaphoreType.DMA] * 4
        ),
    )(x, embed_weight)
